# baseline (speedup 1.0000x reference)
import functools

import jax
import jax.numpy as jnp
from jax import lax
from jax.experimental import pallas as pl
from jax.experimental.pallas import tpu as pltpu
from jax.experimental.pallas import tpu_sc as plsc

NC = 2
NS = 16
NW = NC * NS
CHUNK = 128
LAYERS = 3


def _make_spmm(n, d, epw_chunks, edges_per_worker):
    mesh = plsc.VectorSubcoreMesh(core_axis_name="c", subcore_axis_name="s")
    rows_per_tile = n // NS

    @functools.partial(
        pl.kernel,
        mesh=mesh,
        compiler_params=pltpu.CompilerParams(needs_layout_passes=False),
        out_type=jax.ShapeDtypeStruct((NC, n, d), jnp.float32),
        scratch_types=[
            pltpu.VMEM((3, CHUNK), jnp.int32),
            pltpu.VMEM((3, CHUNK), jnp.int32),
            pltpu.VMEM((CHUNK, d), jnp.float32),
            pltpu.VMEM((CHUNK, d), jnp.float32),
            pltpu.SemaphoreType.DMA,
            pltpu.SemaphoreType.DMA,
            pltpu.SemaphoreType.DMA,
            pltpu.SemaphoreType.DMA,
            pltpu.VMEM_SHARED((n, d), jnp.float32),
        ],
    )
    def spmm(ed_hbm, x_hbm, zeros_hbm, p_hbm,
             ed_a, ed_b, msg_a, msg_b, esem_a, esem_b, gsem_a, gsem_b, acc_sh):
        c = lax.axis_index("c")
        s = lax.axis_index("s")
        wid = s * NC + c

        pltpu.sync_copy(
            zeros_hbm.at[pl.ds(s * rows_per_tile, rows_per_tile)],
            acc_sh.at[pl.ds(s * rows_per_tile, rows_per_tile)],
        )
        plsc.subcore_barrier()

        eds = (ed_a, ed_b)
        msgs = (msg_a, msg_b)
        esems = (esem_a, esem_b)
        gsems = (gsem_a, gsem_b)

        def idx_start(i, j):
            pltpu.async_copy(ed_hbm.at[wid, i], eds[j], esems[j])

        def idx_wait(i, j):
            pltpu.make_async_copy(ed_hbm.at[wid, i], eds[j], esems[j]).wait()

        def gather_start(j):
            pltpu.async_copy(x_hbm.at[eds[j].at[1]], msgs[j], gsems[j])

        def gather_wait(j):
            pltpu.make_async_copy(x_hbm.at[eds[j].at[1]], msgs[j], gsems[j]).wait()

        def compute_scatter(j):
            msg = msgs[j]
            wrow = eds[j].at[2]

            def edge_body(e, carry2):
                widx = jnp.full((16,), e, jnp.int32)
                wvec = plsc.bitcast(plsc.load_gather(wrow, [widx]), jnp.float32)
                for jj in range(d // 16):
                    sl = pl.ds(jj * 16, 16)
                    msg[e, sl] = msg[e, sl] * wvec
                return carry2

            lax.fori_loop(0, CHUNK, edge_body, 0, unroll=2)
            pltpu.sync_copy(msg, acc_sh.at[eds[j].at[0]], add=True)

        idx_start(0, 0)
        idx_start(1, 1)
        idx_wait(0, 0)
        gather_start(0)

        def group_body(t, carry):
            for j in (0, 1):
                i = 2 * t + j
                idx_wait(i + 1, 1 - j)
                gather_start(1 - j)
                gather_wait(j)
                compute_scatter(j)
                idx_start(i + 2, j)
            return carry

        lax.fori_loop(0, epw_chunks // 2 - 1, group_body, 0)
        idx_wait(epw_chunks - 1, 1)
        gather_start(1)
        gather_wait(0)
        compute_scatter(0)
        gather_wait(1)
        compute_scatter(1)

        plsc.subcore_barrier()
        pltpu.sync_copy(
            acc_sh.at[pl.ds(s * rows_per_tile, rows_per_tile)],
            p_hbm.at[c, pl.ds(s * rows_per_tile, rows_per_tile)],
        )

    return spmm


def _make_combine(n, d, scale):
    blk = n // NS
    grid = (n // blk,)

    def body(p_ref, acc_ref, x_ref, accout_ref):
        x = p_ref[0] + p_ref[1]
        x_ref[...] = x
        accout_ref[...] = (acc_ref[...] + x) * scale

    return pl.pallas_call(
        body,
        grid=grid,
        in_specs=[
            pl.BlockSpec((2, blk, d), lambda i: (0, i, 0)),
            pl.BlockSpec((blk, d), lambda i: (i, 0)),
        ],
        out_specs=[
            pl.BlockSpec((blk, d), lambda i: (i, 0)),
            pl.BlockSpec((blk, d), lambda i: (i, 0)),
        ],
        out_shape=[
            jax.ShapeDtypeStruct((n, d), jnp.float32),
            jax.ShapeDtypeStruct((n, d), jnp.float32),
        ],
    )


def kernel(edge_index, edge_values, embedding):
    e = edge_values.shape[0]
    n, d = embedding.shape

    per = NW * CHUNK * 2
    epad = ((e + per - 1) // per) * per
    pad = epad - e
    epw = epad // NW
    rows = jnp.pad(edge_index[0], (0, pad)).reshape(NW, epw // CHUNK, 1, CHUNK)
    cols = jnp.pad(edge_index[1], (0, pad)).reshape(NW, epw // CHUNK, 1, CHUNK)
    vals = jax.lax.bitcast_convert_type(
        jnp.pad(edge_values, (0, pad)), jnp.int32
    ).reshape(NW, epw // CHUNK, 1, CHUNK)
    ed = jnp.concatenate([rows, cols, vals], axis=2)

    align = 8 * NS
    npad = ((n + align - 1) // align) * align
    x0 = jnp.pad(embedding, ((0, npad - n), (0, 0)))
    zeros = jnp.zeros((npad, d), jnp.float32)

    edges_per_worker = epad // NW
    epw_chunks = edges_per_worker // CHUNK

    spmm = _make_spmm(npad, d, epw_chunks, edges_per_worker)

    x = x0
    acc = x0
    for layer in range(LAYERS):
        p = spmm(ed, x, zeros)
        scale = 1.0 / (LAYERS + 1) if layer == LAYERS - 1 else 1.0
        x, acc = _make_combine(npad, d, scale)(p, acc)
    return acc[:n]


# speedup vs baseline: 1.2049x; 1.2049x over previous
import functools

import jax
import jax.numpy as jnp
from jax import lax
from jax.experimental import pallas as pl
from jax.experimental.pallas import tpu as pltpu
from jax.experimental.pallas import tpu_sc as plsc

NC = 2
NS = 16
NW = NC * NS
CHUNK = 128
NSPLIT = 2
LAYERS = 3


def _make_spmm(n, d, epw_chunks):
    mesh = plsc.VectorSubcoreMesh(core_axis_name="c", subcore_axis_name="s")
    rows_per_tile = n // NS
    part = CHUNK // NSPLIT

    @functools.partial(
        pl.kernel,
        mesh=mesh,
        compiler_params=pltpu.CompilerParams(needs_layout_passes=False),
        out_type=jax.ShapeDtypeStruct((NC, n, d), jnp.float32),
        scratch_types=[
            pltpu.VMEM((epw_chunks, CHUNK), jnp.int32),
            pltpu.VMEM((epw_chunks, CHUNK), jnp.int32),
            pltpu.VMEM((epw_chunks, CHUNK), jnp.float32),
            pltpu.VMEM((CHUNK, d), jnp.float32),
            pltpu.SemaphoreType.DMA,
            pltpu.SemaphoreType.DMA,
            pltpu.VMEM_SHARED((n, d), jnp.float32),
        ],
    )
    def spmm(rows_hbm, cols_hbm, vals_hbm, x_hbm, zeros_hbm, p_hbm,
             ridx_v, cidx_v, w_v, msg_v, gsem_a, gsem_b, acc_sh):
        c = lax.axis_index("c")
        s = lax.axis_index("s")
        wid = s * NC + c

        pltpu.sync_copy(rows_hbm.at[wid], ridx_v)
        pltpu.sync_copy(cols_hbm.at[wid], cidx_v)
        pltpu.sync_copy(vals_hbm.at[wid], w_v)

        pltpu.sync_copy(
            zeros_hbm.at[pl.ds(s * rows_per_tile, rows_per_tile)],
            acc_sh.at[pl.ds(s * rows_per_tile, rows_per_tile)],
        )
        plsc.subcore_barrier()

        gsems = (gsem_a, gsem_b)

        def chunk_body(i, carry):
            for k in range(NSPLIT):
                pltpu.async_copy(
                    x_hbm.at[cidx_v.at[i, pl.ds(k * part, part)]],
                    msg_v.at[pl.ds(k * part, part)],
                    gsems[k],
                )
            for k in range(NSPLIT):
                pltpu.make_async_copy(
                    x_hbm.at[cidx_v.at[i, pl.ds(k * part, part)]],
                    msg_v.at[pl.ds(k * part, part)],
                    gsems[k],
                ).wait()

            def edge_body(e, carry2):
                widx = jnp.full((16,), e, jnp.int32)
                wvec = plsc.load_gather(w_v.at[i], [widx])
                for j in range(d // 16):
                    sl = pl.ds(j * 16, 16)
                    msg_v[e, sl] = msg_v[e, sl] * wvec
                return carry2

            lax.fori_loop(0, CHUNK, edge_body, 0, unroll=2)
            pltpu.sync_copy(msg_v, acc_sh.at[ridx_v.at[i]], add=True)
            return carry

        lax.fori_loop(0, epw_chunks, chunk_body, 0)
        plsc.subcore_barrier()
        pltpu.sync_copy(
            acc_sh.at[pl.ds(s * rows_per_tile, rows_per_tile)],
            p_hbm.at[c, pl.ds(s * rows_per_tile, rows_per_tile)],
        )

    return spmm


def _make_combine(n, d, scale):
    blk = n // NS
    grid = (n // blk,)

    def body(p_ref, acc_ref, x_ref, accout_ref):
        x = p_ref[0] + p_ref[1]
        x_ref[...] = x
        accout_ref[...] = (acc_ref[...] + x) * scale

    return pl.pallas_call(
        body,
        grid=grid,
        in_specs=[
            pl.BlockSpec((2, blk, d), lambda i: (0, i, 0)),
            pl.BlockSpec((blk, d), lambda i: (i, 0)),
        ],
        out_specs=[
            pl.BlockSpec((blk, d), lambda i: (i, 0)),
            pl.BlockSpec((blk, d), lambda i: (i, 0)),
        ],
        out_shape=[
            jax.ShapeDtypeStruct((n, d), jnp.float32),
            jax.ShapeDtypeStruct((n, d), jnp.float32),
        ],
    )


def kernel(edge_index, edge_values, embedding):
    e = edge_values.shape[0]
    n, d = embedding.shape

    per = NW * CHUNK
    epad = ((e + per - 1) // per) * per
    pad = epad - e
    epw = epad // NW
    rows = jnp.pad(edge_index[0], (0, pad)).reshape(NW, epw // CHUNK, CHUNK)
    cols = jnp.pad(edge_index[1], (0, pad)).reshape(NW, epw // CHUNK, CHUNK)
    vals = jnp.pad(edge_values, (0, pad)).reshape(NW, epw // CHUNK, CHUNK)

    align = 8 * NS
    npad = ((n + align - 1) // align) * align
    x0 = jnp.pad(embedding, ((0, npad - n), (0, 0)))
    zeros = jnp.zeros((npad, d), jnp.float32)

    epw_chunks = epw // CHUNK
    spmm = _make_spmm(npad, d, epw_chunks)

    x = x0
    acc = x0
    for layer in range(LAYERS):
        p = spmm(rows, cols, vals, x, zeros)
        scale = 1.0 / (LAYERS + 1) if layer == LAYERS - 1 else 1.0
        x, acc = _make_combine(npad, d, scale)(p, acc)
    return acc[:n]
